# baseline (speedup 1.0000x reference)
import functools

import jax
import jax.numpy as jnp
from jax import lax
from jax.experimental import pallas as pl
from jax.experimental.pallas import tpu as pltpu
from jax.experimental.pallas import tpu_sc as plsc

NU = 1000000
NI = 1000000
D = 32
B = 16384

_NC, _NS = 2, 16
_NW = _NC * _NS
_BPW = B // _NW
_CH = 128
_NCHUNK = _BPW // _CH

def _sc_gather_body(ug_hbm, ui_hbm, um_hbm, im_hbm, uidx_hbm, iidx_hbm,
                    out_ug, out_ui, out_um, out_im,
                    idx_u, idx_i, r_ug, r_ui, r_um, r_im, sem):
    wid = lax.axis_index("s") * _NC + lax.axis_index("c")
    base = wid * _BPW
    crow = wid * _NCHUNK

    pltpu.sync_copy(uidx_hbm.at[pl.ds(crow, _NCHUNK)], idx_u.at[pl.ds(crow, _NCHUNK)])
    pltpu.sync_copy(iidx_hbm.at[pl.ds(crow, _NCHUNK)], idx_i.at[pl.ds(crow, _NCHUNK)])

    copies = []
    for j in range(_NCHUNK):
        dst = pl.ds(j * _CH, _CH)
        copies.append(pltpu.async_copy(ug_hbm.at[idx_u.at[crow + j]], r_ug.at[dst], sem))
        copies.append(pltpu.async_copy(ui_hbm.at[idx_i.at[crow + j]], r_ui.at[dst], sem))
        copies.append(pltpu.async_copy(um_hbm.at[idx_u.at[crow + j]], r_um.at[dst], sem))
        copies.append(pltpu.async_copy(im_hbm.at[idx_i.at[crow + j]], r_im.at[dst], sem))
    for c in copies:
        c.wait()

    slab = pl.ds(base, _BPW)
    pltpu.sync_copy(r_ug, out_ug.at[slab])
    pltpu.sync_copy(r_ui, out_ui.at[slab])
    pltpu.sync_copy(r_um, out_um.at[slab])
    pltpu.sync_copy(r_im, out_im.at[slab])


@functools.lru_cache(maxsize=1)
def _sc_gather():
    mesh = plsc.VectorSubcoreMesh(core_axis_name="c", subcore_axis_name="s",
                                  num_cores=_NC, num_subcores=_NS)
    return pl.kernel(
        _sc_gather_body,
        out_type=[jax.ShapeDtypeStruct((B, D), jnp.float32) for _ in range(4)],
        mesh=mesh,
        scratch_types=[
            pltpu.VMEM((B // _CH, _CH), jnp.int32),
            pltpu.VMEM((B // _CH, _CH), jnp.int32),
            pltpu.VMEM((_BPW, D), jnp.float32),
            pltpu.VMEM((_BPW, D), jnp.float32),
            pltpu.VMEM((_BPW, D), jnp.float32),
            pltpu.VMEM((_BPW, D), jnp.float32),
            pltpu.SemaphoreType.DMA,
        ],
        compiler_params=pltpu.CompilerParams(use_tc_tiling_on_sc=False),
    )


def _tc_body(ug, ui, um, im, w0t, b0, w1t, b1, w2t, b2, w3t, b3, wpg, wpm, bp,
             out):
    gmf = ug[...] * ui[...]
    w0 = w0t[...]
    h = jnp.maximum(um[...] @ w0[:D] + im[...] @ w0[D:] + b0[...], 0.0)
    h = jnp.maximum(h @ w1t[...] + b1[...], 0.0)
    h = jnp.maximum(h @ w2t[...] + b2[...], 0.0)
    h = jnp.maximum(h @ w3t[...] + b3[...], 0.0)
    p = gmf @ wpg[...] + h @ wpm[...] + bp[...]
    out[...] = 1.0 / (1.0 + jnp.exp(-p))


def kernel(user_indices, item_indices, embed_user_gmf, embed_item_gmf,
           embed_user_mlp, embed_item_mlp, W0, b0, W1, b1, W2, b2, W3, b3,
           Wp, bp):
    uidx = user_indices.astype(jnp.int32).reshape(B // _CH, _CH)
    iidx = item_indices.astype(jnp.int32).reshape(B // _CH, _CH)

    ug, ui, um, im = _sc_gather()(embed_user_gmf, embed_item_gmf,
                                  embed_user_mlp, embed_item_mlp, uidx, iidx)

    blk = 2048
    grid = B // blk
    batch_spec = pl.BlockSpec((blk, D), lambda i: (i, 0))

    def full(shape):
        return pl.BlockSpec(shape, lambda i: tuple(0 for _ in shape))

    w0t = W0.T
    w1t = W1.T
    w2t = W2.T
    w3t = W3.T
    wpg = Wp[:, :D].T
    wpm = Wp[:, D:].T

    out = pl.pallas_call(
        _tc_body,
        grid=(grid,),
        in_specs=[
            batch_spec, batch_spec, batch_spec, batch_spec,
            full((2 * D, 2 * D)), full((1, 2 * D)),
            full((2 * D, 32)), full((1, 32)),
            full((32, 16)), full((1, 16)),
            full((16, 8)), full((1, 8)),
            full((D, 1)), full((8, 1)), full((1, 1)),
        ],
        out_specs=pl.BlockSpec((blk, 1), lambda i: (i, 0)),
        out_shape=jax.ShapeDtypeStruct((B, 1), jnp.float32),
    )(ug, ui, um, im,
      w0t, b0.reshape(1, -1), w1t, b1.reshape(1, -1), w2t, b2.reshape(1, -1),
      w3t, b3.reshape(1, -1), wpg, wpm, bp.reshape(1, 1))

    return out.reshape(B)
